# baseline (speedup 1.0000x reference)
import jax
import jax.numpy as jnp
from jax.experimental import pallas as pl

N = 131072
D_IN = 512
D_OUT = 32
BLK = 8192
NSTREAM = 4
SUB = BLK // NSTREAM


def _body(*refs):
    x_refs = refs[:NSTREAM]
    wt_ref, b_ref, o_ref = refs[NSTREAM:]
    for q in range(NSTREAM):
        acc = jax.lax.dot_general(
            x_refs[q][:], wt_ref[:],
            (((1,), (0,)), ((), ())),
            preferred_element_type=jnp.float32,
        )
        o_ref[pl.ds(q * SUB, SUB), :] = jnp.maximum(acc + b_ref[:], 0.0)


def kernel(x, W1, b1):
    wt = W1.T
    grid = (N // BLK,)
    in_specs = [
        pl.BlockSpec((SUB, D_IN), lambda i, q=q: (NSTREAM * i + q, 0))
        for q in range(NSTREAM)
    ] + [
        pl.BlockSpec((D_IN, D_OUT), lambda i: (0, 0)),
        pl.BlockSpec((D_OUT,), lambda i: (0,)),
    ]
    return pl.pallas_call(
        _body,
        grid=grid,
        in_specs=in_specs,
        out_specs=pl.BlockSpec((BLK, D_OUT), lambda i: (i, 0)),
        out_shape=jax.ShapeDtypeStruct((N, D_OUT), jnp.float32),
    )(*([x] * NSTREAM), wt, b1)
